# Initial kernel scaffold; baseline (speedup 1.0000x reference)
import jax
import jax.numpy as jnp
from jax.experimental import pallas as pl


def kernel(x, edge_index, W1, al1, ar1, b1, W2, al2, ar2, b2, fcW, fcb):
    raise NotImplementedError("write your pallas kernel here")


# speedup vs baseline: 23.5408x; 23.5408x over previous
import functools

import jax
import jax.numpy as jnp
from jax import lax
from jax.experimental import pallas as pl
from jax.experimental.pallas import tpu as pltpu
from jax.experimental.pallas import tpu_sc as plsc

N = 10000
E = 320000
D = 128
H = 128

NP = 10240
NW = 32
EPT = E // NW
K = 80
NCH = EPT // K
NQ = H // 16
NPT = NP // 16
BLK = 1280
GRID = NP // BLK
PB = BLK // H

_mesh = plsc.VectorSubcoreMesh(core_axis_name="c", subcore_axis_name="s")


@functools.partial(
    pl.kernel,
    mesh=_mesh,
    compiler_params=pltpu.CompilerParams(needs_layout_passes=False),
    out_type=[
        jax.ShapeDtypeStruct((2, NP, H), jnp.float32),
        jax.ShapeDtypeStruct((2, 1, NP), jnp.float32),
    ],
    scratch_types=[
        pltpu.VMEM((NP,), jnp.float32),
        pltpu.VMEM((NP,), jnp.float32),
        pltpu.VMEM((2, K), jnp.int32),
        pltpu.VMEM((K,), jnp.float32),
        pltpu.VMEM((K, H), jnp.float32),
        pltpu.VMEM_SHARED((NP, H), jnp.float32),
        pltpu.VMEM_SHARED((NP,), jnp.float32),
        pltpu.VMEM((NPT,), jnp.float32),
        pltpu.SemaphoreType.DMA,
    ],
)
def _sc_layer(ei_hbm, elr_hbm, fe_hbm, s_out, d_out,
              el_v, er_v, eb_v, ee_v, rows_v, acc_sh, den_sh, zb_v,
              sem):
    cid = lax.axis_index("c")
    sid = lax.axis_index("s")
    wid = cid * 16 + sid

    pltpu.sync_copy(elr_hbm.at[0], el_v)
    pltpu.sync_copy(elr_hbm.at[1], er_v)

    def _zero_row(i, _):
        for q in range(NQ):
            rows_v[i, pl.ds(q * 16, 16)] = jnp.zeros((16,), jnp.float32)
        return 0
    lax.fori_loop(0, K, _zero_row, 0)

    def _zero_den(i, _):
        zb_v[pl.ds(i * 16, 16)] = jnp.zeros((16,), jnp.float32)
        return 0
    lax.fori_loop(0, NPT // 16, _zero_den, 0)
    for j in range(NPT // K):
        pltpu.sync_copy(rows_v, acc_sh.at[pl.ds(sid * NPT + j * K, K)])
    pltpu.sync_copy(zb_v, den_sh.at[pl.ds(sid * NPT, NPT)])
    plsc.subcore_barrier()

    def _chunk(ch, _):
        pltpu.sync_copy(ei_hbm.at[wid, ch], eb_v)
        cp = pltpu.async_copy(fe_hbm.at[eb_v.at[0]], rows_v, sem)
        for g in range(K // 16):
            sv = eb_v[0, pl.ds(g * 16, 16)]
            dv = eb_v[1, pl.ds(g * 16, 16)]
            e = plsc.load_gather(el_v, [sv]) + plsc.load_gather(er_v, [dv])
            e = jnp.where(e > 0, e, 0.2 * e)
            ee_v[pl.ds(g * 16, 16)] = jnp.exp(e)
        cp.wait()

        def _scale(i, _):
            sp = plsc.load_gather(ee_v, [jnp.full((16,), i, jnp.int32)])
            for q in range(NQ):
                rows_v[i, pl.ds(q * 16, 16)] = rows_v[i, pl.ds(q * 16, 16)] * sp
            return 0
        lax.fori_loop(0, K, _scale, 0)

        pltpu.sync_copy(rows_v, acc_sh.at[eb_v.at[1]], add=True)
        pltpu.sync_copy(ee_v, den_sh.at[eb_v.at[1]], add=True)
        return 0

    lax.fori_loop(0, NCH, _chunk, 0)
    plsc.subcore_barrier()

    for j in range(NPT // K):
        off = sid * NPT + j * K
        pltpu.sync_copy(acc_sh.at[pl.ds(off, K)], s_out.at[cid, pl.ds(off, K)])
    pltpu.sync_copy(den_sh.at[pl.ds(sid * NPT, NPT)],
                    d_out.at[cid, 0, pl.ds(sid * NPT, NPT)])


def _dots(feat, al, ar, elr_ref):
    elr_ref[0:1, :] = jnp.sum(feat * al, axis=1)[None, :]
    elr_ref[1:2, :] = jnp.sum(feat * ar, axis=1)[None, :]


def _tc_in_body(h_ref, w_ref, al_ref, ar_ref, fe_ref, elr_ref):
    feat = jnp.dot(h_ref[...], w_ref[...], preferred_element_type=jnp.float32)
    fe_ref[...] = feat
    _dots(feat, al_ref[...], ar_ref[...], elr_ref)


def _merge(s_ref, d_ref):
    i = pl.program_id(0)
    s = s_ref[...]
    hs = s[0] + s[1]
    p = d_ref[0, :, :] + d_ref[1, :, :]
    nr = NP // H
    r80 = (lax.broadcasted_iota(jnp.int32, (BLK, nr), 0) + i * BLK) // H
    k80 = lax.broadcasted_iota(jnp.int32, (BLK, nr), 1)
    sel = jnp.where(r80 == k80, 1.0, 0.0).astype(jnp.float32)
    q = jnp.dot(sel, p, preferred_element_type=jnp.float32)
    rl = lax.broadcasted_iota(jnp.int32, (BLK, H), 0) % H
    cl = lax.broadcasted_iota(jnp.int32, (BLK, H), 1)
    den = jnp.sum(jnp.where(rl == cl, q, 0.0), axis=1)
    return hs, den


def _tc_mid_body(s_ref, d_ref, b_ref, w_ref, al_ref, ar_ref, fe_ref, elr_ref):
    hs, den = _merge(s_ref, d_ref)
    h = jnp.maximum(hs / (den + 1e-9)[:, None] + b_ref[...], 0.0)
    feat = jnp.dot(h, w_ref[...], preferred_element_type=jnp.float32)
    fe_ref[...] = feat
    _dots(feat, al_ref[...], ar_ref[...], elr_ref)


def _tc_out_body(s_ref, d_ref, b_ref, fcw_ref, fcb_ref, out_ref, acc_ref):
    i = pl.program_id(0)
    hs, den = _merge(s_ref, d_ref)
    h = jnp.maximum(hs / (den + 1e-9)[:, None] + b_ref[...], 0.0)
    rows = lax.broadcasted_iota(jnp.int32, (BLK, 1), 0) + i * BLK
    h = jnp.where(rows < N, h, 0.0)
    psum = jnp.sum(h, axis=0, keepdims=True)

    @pl.when(i == 0)
    def _():
        acc_ref[...] = psum

    @pl.when(i > 0)
    def _():
        acc_ref[...] = acc_ref[...] + psum

    @pl.when(i == GRID - 1)
    def _():
        hg = acc_ref[...] * (1.0 / N)
        out_ref[...] = (jnp.dot(hg, fcw_ref[...],
                                preferred_element_type=jnp.float32)
                        + fcb_ref[...])


_tc_in = pl.pallas_call(
    _tc_in_body,
    grid=(GRID,),
    in_specs=[
        pl.BlockSpec((BLK, D), lambda i: (i, 0)),
        pl.BlockSpec((D, H), lambda i: (0, 0)),
        pl.BlockSpec((1, H), lambda i: (0, 0)),
        pl.BlockSpec((1, H), lambda i: (0, 0)),
    ],
    out_specs=[
        pl.BlockSpec((BLK, H), lambda i: (i, 0)),
        pl.BlockSpec((8, BLK), lambda i: (0, i)),
    ],
    out_shape=[
        jax.ShapeDtypeStruct((NP, H), jnp.float32),
        jax.ShapeDtypeStruct((8, NP), jnp.float32),
    ],
)

_tc_mid = pl.pallas_call(
    _tc_mid_body,
    grid=(GRID,),
    in_specs=[
        pl.BlockSpec((2, BLK, H), lambda i: (0, i, 0)),
        pl.BlockSpec((2, NP // H, H), lambda i: (0, 0, 0)),
        pl.BlockSpec((1, H), lambda i: (0, 0)),
        pl.BlockSpec((H, H), lambda i: (0, 0)),
        pl.BlockSpec((1, H), lambda i: (0, 0)),
        pl.BlockSpec((1, H), lambda i: (0, 0)),
    ],
    out_specs=[
        pl.BlockSpec((BLK, H), lambda i: (i, 0)),
        pl.BlockSpec((8, BLK), lambda i: (0, i)),
    ],
    out_shape=[
        jax.ShapeDtypeStruct((NP, H), jnp.float32),
        jax.ShapeDtypeStruct((8, NP), jnp.float32),
    ],
)

_tc_out = pl.pallas_call(
    _tc_out_body,
    grid=(GRID,),
    in_specs=[
        pl.BlockSpec((2, BLK, H), lambda i: (0, i, 0)),
        pl.BlockSpec((2, NP // H, H), lambda i: (0, 0, 0)),
        pl.BlockSpec((1, H), lambda i: (0, 0)),
        pl.BlockSpec((H, 1), lambda i: (0, 0)),
        pl.BlockSpec((1, 1), lambda i: (0, 0)),
    ],
    out_specs=pl.BlockSpec((1, 1), lambda i: (0, 0)),
    out_shape=jax.ShapeDtypeStruct((1, 1), jnp.float32),
    scratch_shapes=[pltpu.VMEM((1, H), jnp.float32)],
)


def kernel(x, edge_index, W1, al1, ar1, b1, W2, al2, ar2, b2, fcW, fcb):
    ei = edge_index.reshape(2, NW, NCH, K).transpose(1, 2, 0, 3)
    xp = jnp.pad(x, ((0, NP - N), (0, 0)))

    fe1, elr1 = _tc_in(xp, W1, al1, ar1)
    s1, d1 = _sc_layer(ei, elr1, fe1)
    fe2, elr2 = _tc_mid(s1, d1.reshape(2, NP // H, H), b1, W2, al2, ar2)
    s2, d2 = _sc_layer(ei, elr2, fe2)
    out = _tc_out(s2, d2.reshape(2, NP // H, H), b2, fcW, fcb.reshape(1, 1))
    return out


# speedup vs baseline: 47.4409x; 2.0153x over previous
import functools

import jax
import jax.numpy as jnp
from jax import lax
from jax.experimental import pallas as pl
from jax.experimental.pallas import tpu as pltpu
from jax.experimental.pallas import tpu_sc as plsc

N = 10000
E = 320000
D = 128
H = 128

NP = 10240
NW = 32
EPT = E // NW
K = 80
NCH = EPT // K
NQ = H // 16
NPT = NP // 16
BLK = 1280
GRID = NP // BLK
PB = BLK // H

_mesh = plsc.VectorSubcoreMesh(core_axis_name="c", subcore_axis_name="s")


@functools.partial(
    pl.kernel,
    mesh=_mesh,
    compiler_params=pltpu.CompilerParams(needs_layout_passes=False),
    out_type=[
        jax.ShapeDtypeStruct((2, NP, H), jnp.float32),
        jax.ShapeDtypeStruct((2, 1, NP), jnp.float32),
    ],
    scratch_types=[
        pltpu.VMEM((NP,), jnp.float32),
        pltpu.VMEM((NP,), jnp.float32),
        pltpu.VMEM((2, 2, K), jnp.int32),
        pltpu.VMEM((2, K), jnp.int32),
        pltpu.VMEM((2 * K,), jnp.float32),
        pltpu.VMEM((2, K, H), jnp.float32),
        pltpu.VMEM_SHARED((NP, H), jnp.float32),
        pltpu.VMEM_SHARED((NP,), jnp.float32),
        pltpu.VMEM((NPT,), jnp.float32),
        pltpu.SemaphoreType.DMA,
        pltpu.SemaphoreType.DMA,
        pltpu.SemaphoreType.DMA,
        pltpu.SemaphoreType.DMA,
    ],
)
def _sc_layer(ei_hbm, elr_hbm, fe_hbm, s_out, d_out,
              el_v, er_v, eb_v, sidx_v, ee_v, rows_v, acc_sh, den_sh, zb_v,
              semi, semg, sems, semd):
    cid = lax.axis_index("c")
    sid = lax.axis_index("s")
    wid = cid * 16 + sid

    pltpu.sync_copy(elr_hbm.at[0], el_v)
    pltpu.sync_copy(elr_hbm.at[1], er_v)

    def _zero_row(i, _):
        for q in range(NQ):
            rows_v[0, i, pl.ds(q * 16, 16)] = jnp.zeros((16,), jnp.float32)
        return 0
    lax.fori_loop(0, K, _zero_row, 0, unroll=4)

    def _zero_den(i, _):
        zb_v[pl.ds(i * 16, 16)] = jnp.zeros((16,), jnp.float32)
        return 0
    lax.fori_loop(0, NPT // 16, _zero_den, 0, unroll=4)
    for j in range(NPT // K):
        pltpu.sync_copy(rows_v.at[0], acc_sh.at[pl.ds(sid * NPT + j * K, K)])
    pltpu.sync_copy(zb_v, den_sh.at[pl.ds(sid * NPT, NPT)])
    plsc.subcore_barrier()

    def _ee_compute(p, ch):
        for g in range(K // 16):
            sv = eb_v[p, 0, pl.ds(g * 16, 16)]
            dv = eb_v[p, 1, pl.ds(g * 16, 16)]
            e = plsc.load_gather(el_v, [sv]) + plsc.load_gather(er_v, [dv])
            e = jnp.where(e > 0, e, 0.2 * e)
            ee_v[pl.ds(p * K + g * 16, 16)] = jnp.exp(e)
            sidx_v[p, pl.ds(g * 16, 16)] = dv

    def _scale_rows(p):
        def _scale(i, _):
            sp = plsc.load_gather(ee_v, [jnp.full((16,), p * K, jnp.int32) + i])
            for q in range(NQ):
                rows_v[p, i, pl.ds(q * 16, 16)] = (
                    rows_v[p, i, pl.ds(q * 16, 16)] * sp)
            return 0
        lax.fori_loop(0, K, _scale, 0, unroll=4)

    def _wait_scatters(p):
        pltpu.make_async_copy(
            rows_v.at[p], acc_sh.at[sidx_v.at[p]], sems).wait()
        pltpu.make_async_copy(
            ee_v.at[pl.ds(p * K, K)], den_sh.at[sidx_v.at[p]], semd).wait()

    def _issue_scatters(p):
        pltpu.async_copy(rows_v.at[p], acc_sh.at[sidx_v.at[p]], sems,
                         add=True)
        pltpu.async_copy(ee_v.at[pl.ds(p * K, K)], den_sh.at[sidx_v.at[p]],
                         semd, add=True)

    pltpu.sync_copy(ei_hbm.at[wid, 0], eb_v.at[0])
    pltpu.async_copy(fe_hbm.at[eb_v.at[0, 0]], rows_v.at[0], semg)
    pltpu.async_copy(ei_hbm.at[wid, 1], eb_v.at[1], semi)

    def _pair(g2, _):
        for p in range(2):
            ch = g2 * 2 + p
            _ee_compute(p, ch)

            @pl.when(ch >= 1)
            def _():
                _wait_scatters(1 - p)

            pltpu.make_async_copy(ei_hbm.at[wid, ch], eb_v.at[1 - p],
                                  semi).wait()
            pltpu.async_copy(fe_hbm.at[eb_v.at[1 - p, 0]], rows_v.at[1 - p],
                             semg)
            pltpu.make_async_copy(fe_hbm.at[eb_v.at[p, 0]], rows_v.at[p],
                                  semg).wait()

            @pl.when(ch + 2 < NCH)
            def _():
                pltpu.async_copy(ei_hbm.at[wid, ch + 2], eb_v.at[p], semi)

            _scale_rows(p)
            _issue_scatters(p)
        return 0

    lax.fori_loop(0, (NCH - 1) // 2, _pair, 0)

    lp = (NCH - 1) % 2
    _ee_compute(lp, NCH - 1)
    _wait_scatters(1 - lp)
    pltpu.make_async_copy(fe_hbm.at[eb_v.at[lp, 0]], rows_v.at[lp],
                          semg).wait()
    _scale_rows(lp)
    pltpu.sync_copy(rows_v.at[lp], acc_sh.at[sidx_v.at[lp]], add=True)
    pltpu.sync_copy(ee_v.at[pl.ds(lp * K, K)], den_sh.at[sidx_v.at[lp]],
                    add=True)
    plsc.subcore_barrier()

    for j in range(NPT // K):
        off = sid * NPT + j * K
        pltpu.sync_copy(acc_sh.at[pl.ds(off, K)], s_out.at[cid, pl.ds(off, K)])
    pltpu.sync_copy(den_sh.at[pl.ds(sid * NPT, NPT)],
                    d_out.at[cid, 0, pl.ds(sid * NPT, NPT)])


def _dots(feat, al, ar, elr_ref):
    elr_ref[0:1, :] = jnp.sum(feat * al, axis=1)[None, :]
    elr_ref[1:2, :] = jnp.sum(feat * ar, axis=1)[None, :]


def _tc_in_body(h_ref, w_ref, al_ref, ar_ref, fe_ref, elr_ref):
    feat = jnp.dot(h_ref[...], w_ref[...], preferred_element_type=jnp.float32)
    fe_ref[...] = feat
    _dots(feat, al_ref[...], ar_ref[...], elr_ref)


def _merge(s_ref, d_ref):
    i = pl.program_id(0)
    s = s_ref[...]
    hs = s[0] + s[1]
    p = d_ref[0, :, :] + d_ref[1, :, :]
    nr = NP // H
    r80 = (lax.broadcasted_iota(jnp.int32, (BLK, nr), 0) + i * BLK) // H
    k80 = lax.broadcasted_iota(jnp.int32, (BLK, nr), 1)
    sel = jnp.where(r80 == k80, 1.0, 0.0).astype(jnp.float32)
    q = jnp.dot(sel, p, preferred_element_type=jnp.float32)
    rl = lax.broadcasted_iota(jnp.int32, (BLK, H), 0) % H
    cl = lax.broadcasted_iota(jnp.int32, (BLK, H), 1)
    den = jnp.sum(jnp.where(rl == cl, q, 0.0), axis=1)
    return hs, den


def _tc_mid_body(s_ref, d_ref, b_ref, w_ref, al_ref, ar_ref, fe_ref, elr_ref):
    hs, den = _merge(s_ref, d_ref)
    h = jnp.maximum(hs / (den + 1e-9)[:, None] + b_ref[...], 0.0)
    feat = jnp.dot(h, w_ref[...], preferred_element_type=jnp.float32)
    fe_ref[...] = feat
    _dots(feat, al_ref[...], ar_ref[...], elr_ref)


def _tc_out_body(s_ref, d_ref, b_ref, fcw_ref, fcb_ref, out_ref, acc_ref):
    i = pl.program_id(0)
    hs, den = _merge(s_ref, d_ref)
    h = jnp.maximum(hs / (den + 1e-9)[:, None] + b_ref[...], 0.0)
    rows = lax.broadcasted_iota(jnp.int32, (BLK, 1), 0) + i * BLK
    h = jnp.where(rows < N, h, 0.0)
    psum = jnp.sum(h, axis=0, keepdims=True)

    @pl.when(i == 0)
    def _():
        acc_ref[...] = psum

    @pl.when(i > 0)
    def _():
        acc_ref[...] = acc_ref[...] + psum

    @pl.when(i == GRID - 1)
    def _():
        hg = acc_ref[...] * (1.0 / N)
        out_ref[...] = (jnp.dot(hg, fcw_ref[...],
                                preferred_element_type=jnp.float32)
                        + fcb_ref[...])


_tc_in = pl.pallas_call(
    _tc_in_body,
    grid=(GRID,),
    in_specs=[
        pl.BlockSpec((BLK, D), lambda i: (i, 0)),
        pl.BlockSpec((D, H), lambda i: (0, 0)),
        pl.BlockSpec((1, H), lambda i: (0, 0)),
        pl.BlockSpec((1, H), lambda i: (0, 0)),
    ],
    out_specs=[
        pl.BlockSpec((BLK, H), lambda i: (i, 0)),
        pl.BlockSpec((8, BLK), lambda i: (0, i)),
    ],
    out_shape=[
        jax.ShapeDtypeStruct((NP, H), jnp.float32),
        jax.ShapeDtypeStruct((8, NP), jnp.float32),
    ],
)

_tc_mid = pl.pallas_call(
    _tc_mid_body,
    grid=(GRID,),
    in_specs=[
        pl.BlockSpec((2, BLK, H), lambda i: (0, i, 0)),
        pl.BlockSpec((2, NP // H, H), lambda i: (0, 0, 0)),
        pl.BlockSpec((1, H), lambda i: (0, 0)),
        pl.BlockSpec((H, H), lambda i: (0, 0)),
        pl.BlockSpec((1, H), lambda i: (0, 0)),
        pl.BlockSpec((1, H), lambda i: (0, 0)),
    ],
    out_specs=[
        pl.BlockSpec((BLK, H), lambda i: (i, 0)),
        pl.BlockSpec((8, BLK), lambda i: (0, i)),
    ],
    out_shape=[
        jax.ShapeDtypeStruct((NP, H), jnp.float32),
        jax.ShapeDtypeStruct((8, NP), jnp.float32),
    ],
)

_tc_out = pl.pallas_call(
    _tc_out_body,
    grid=(GRID,),
    in_specs=[
        pl.BlockSpec((2, BLK, H), lambda i: (0, i, 0)),
        pl.BlockSpec((2, NP // H, H), lambda i: (0, 0, 0)),
        pl.BlockSpec((1, H), lambda i: (0, 0)),
        pl.BlockSpec((H, 1), lambda i: (0, 0)),
        pl.BlockSpec((1, 1), lambda i: (0, 0)),
    ],
    out_specs=pl.BlockSpec((1, 1), lambda i: (0, 0)),
    out_shape=jax.ShapeDtypeStruct((1, 1), jnp.float32),
    scratch_shapes=[pltpu.VMEM((1, H), jnp.float32)],
)


def kernel(x, edge_index, W1, al1, ar1, b1, W2, al2, ar2, b2, fcW, fcb):
    ei = edge_index.reshape(2, NW, NCH, K).transpose(1, 2, 0, 3)
    xp = jnp.pad(x, ((0, NP - N), (0, 0)))

    fe1, elr1 = _tc_in(xp, W1, al1, ar1)
    s1, d1 = _sc_layer(ei, elr1, fe1)
    fe2, elr2 = _tc_mid(s1, d1.reshape(2, NP // H, H), b1, W2, al2, ar2)
    s2, d2 = _sc_layer(ei, elr2, fe2)
    out = _tc_out(s2, d2.reshape(2, NP // H, H), b2, fcW, fcb.reshape(1, 1))
    return out
